# baseline (speedup 1.0000x reference)
import functools

import jax
import jax.numpy as jnp
from jax import lax
from jax.experimental import pallas as pl
from jax.experimental.pallas import tpu as pltpu
from jax.experimental.pallas import tpu_sc as plsc

_N = 10000
_E = 320000
_D = 128
_NC = 2
_NS = 16
_NW = _NC * _NS
_EPW = _E // _NW
_CHUNK = 80
_NCHUNK = _EPW // _CHUNK
_NP = 10240
_RPT = _NP // _NS
_LANES = 16


def _sc_message(src, dst, x, ea, aff):
    mesh = plsc.VectorSubcoreMesh(core_axis_name="c", subcore_axis_name="s")

    @functools.partial(
        pl.kernel,
        mesh=mesh,
        out_type=jax.ShapeDtypeStruct((_NC * _NP, _D), jnp.float32),
        scratch_types=(
            [pltpu.VMEM_SHARED((_NP, _D), jnp.float32)]
            + [pltpu.VMEM((2 * _D,), jnp.float32)]
            + [pltpu.VMEM((_CHUNK,), jnp.int32)] * 2
            + [pltpu.VMEM((_CHUNK,), jnp.int32)] * 2
            + [pltpu.VMEM((_CHUNK, _D), jnp.float32)] * 4
            + [pltpu.SemaphoreType.DMA] * 11
        ),
    )
    def body(src_hbm, dst_hbm, x_hbm, ea_hbm, aff_hbm, out_hbm,
             acc, affb, sx0, sx1, dx0, dx1,
             xv0, xv1, ev0, ev1,
             p0, p1, q0, q1, g0, g1,
             f0, f1, s0, s1, zsem):
        sxs = (sx0, sx1)
        dxs = (dx0, dx1)
        xvs = (xv0, xv1)
        evs = (ev0, ev1)
        psems = (p0, p1)
        qsems = (q0, q1)
        gsems = (g0, g1)
        fsems = (f0, f1)
        ssems = (s0, s1)
        cid = lax.axis_index("c")
        sid = lax.axis_index("s")
        wid = sid * _NC + cid
        ebase = wid * _EPW

        zv = jnp.zeros((_LANES,), jnp.float32)

        @plsc.parallel_loop(0, _CHUNK, unroll=4)
        def _(r):
            for c in range(_D // _LANES):
                xv0[r, pl.ds(c * _LANES, _LANES)] = zv

        row0 = sid * _RPT
        for k in range(_RPT // _CHUNK):
            pltpu.async_copy(
                xv0, acc.at[pl.ds(row0 + k * _CHUNK, _CHUNK)], zsem)
        pltpu.sync_copy(aff_hbm, affb)
        for k in range(_RPT // _CHUNK):
            pltpu.make_async_copy(
                xv0, acc.at[pl.ds(0, _CHUNK)], zsem).wait()
        plsc.subcore_barrier()

        scv = [affb[pl.ds(c * _LANES, _LANES)] for c in range(_D // _LANES)]
        shv = [affb[pl.ds(_D + c * _LANES, _LANES)]
               for c in range(_D // _LANES)]

        def sfetch(j, b):
            pltpu.async_copy(
                src_hbm.at[pl.ds(ebase + j * _CHUNK, _CHUNK)], sxs[b], psems[b])

        def wait_sidx(b):
            pltpu.make_async_copy(
                src_hbm.at[pl.ds(0, _CHUNK)], sxs[b], psems[b]).wait()

        def dfetch(j, b):
            pltpu.async_copy(
                dst_hbm.at[pl.ds(ebase + j * _CHUNK, _CHUNK)], dxs[b], qsems[b])

        def wait_didx(b):
            pltpu.make_async_copy(
                dst_hbm.at[pl.ds(0, _CHUNK)], dxs[b], qsems[b]).wait()

        def issue(j, b):
            pltpu.async_copy(x_hbm.at[sxs[b]], xvs[b], gsems[b])
            pltpu.async_copy(
                ea_hbm.at[pl.ds(ebase + j * _CHUNK, _CHUNK)], evs[b], fsems[b])

        def wait_in(b):
            pltpu.make_async_copy(
                x_hbm.at[pl.ds(0, _CHUNK)], xvs[b], gsems[b]).wait()
            pltpu.make_async_copy(
                ea_hbm.at[pl.ds(0, _CHUNK)], evs[b], fsems[b]).wait()

        def wait_sc(b):
            pltpu.make_async_copy(
                evs[b], acc.at[pl.ds(0, _CHUNK)], ssems[b]).wait()

        def compute(b):
            xs, ms = xvs[b], evs[b]

            def rowf(r, carry):
                for c in range(_D // _LANES):
                    sl = pl.ds(c * _LANES, _LANES)
                    ms[r, sl] = jnp.maximum(
                        xs[r, sl] * scv[c] + (ms[r, sl] + shv[c]), 0.0)
                return carry

            lax.fori_loop(0, _CHUNK, rowf, 0)

        def scatter(j, b):
            pltpu.async_copy(evs[b], acc.at[dxs[b]], ssems[b], add=True)

        sfetch(0, 0)
        sfetch(1, 1)
        dfetch(0, 0)
        wait_sidx(0)
        issue(0, 0)

        def step(j, b):
            o = 1 - b
            wait_in(b)

            @pl.when(j + 2 < _NCHUNK)
            def _():
                sfetch(j + 2, b)

            @pl.when(j >= 1)
            def _():
                wait_sc(o)

            @pl.when(j + 1 < _NCHUNK)
            def _():
                wait_sidx(o)
                issue(j + 1, o)
                dfetch(j + 1, o)

            compute(b)
            wait_didx(b)
            scatter(j, b)

        def pair(q, carry):
            step(2 * q, 0)
            step(2 * q + 1, 1)
            return carry

        lax.fori_loop(0, _NCHUNK // 2, pair, 0)

        for j in range(2 * (_NCHUNK // 2), _NCHUNK):
            step(j, j % 2)
        wait_sc((_NCHUNK - 1) % 2)

        plsc.subcore_barrier()
        pltpu.sync_copy(acc.at[pl.ds(row0, _RPT)],
                        out_hbm.at[pl.ds(cid * _NP + row0, _RPT)])

    return body(src, dst, x, ea, aff)


_BR = 1000
_NB = _N // _BR


def _tc_linear(x, p, W, b, g, be, aff):
    def body(x_ref, p0_ref, p1_ref, w_ref, b_ref, g_ref, be_ref, aff_ref,
             y_ref, affo_ref, s1, s2):
        i = pl.program_id(0)
        h = (x_ref[...] * aff_ref[0] + aff_ref[1]
             + p0_ref[...] + p1_ref[...])
        y = jnp.dot(h, w_ref[...], preferred_element_type=jnp.float32)
        y = jnp.maximum(y + b_ref[...], 0.0)
        y_ref[...] = y
        c1 = jnp.sum(y, axis=0).reshape(1, _D)
        c2 = jnp.sum(y * y, axis=0).reshape(1, _D)

        @pl.when(i == 0)
        def _():
            s1[...] = c1
            s2[...] = c2
            affo_ref[...] = jnp.zeros((8, _D), jnp.float32)

        @pl.when(i > 0)
        def _():
            s1[...] = s1[...] + c1
            s2[...] = s2[...] + c2

        @pl.when(i == _NB - 1)
        def _():
            mean = s1[0] / _N
            var = s2[0] / _N - mean * mean
            scale = g_ref[0] * lax.rsqrt(var + 1e-5)
            shift = be_ref[0] - mean * scale
            affo_ref[0, :] = scale
            affo_ref[1, :] = shift

    return pl.pallas_call(
        body,
        grid=(_NB,),
        in_specs=[
            pl.BlockSpec((_BR, _D), lambda i: (i, 0)),
            pl.BlockSpec((_BR, _D), lambda i: (i, 0)),
            pl.BlockSpec((_BR, _D), lambda i: (i, 0)),
            pl.BlockSpec((_D, _D), lambda i: (0, 0)),
            pl.BlockSpec((1, _D), lambda i: (0, 0)),
            pl.BlockSpec((1, _D), lambda i: (0, 0)),
            pl.BlockSpec((1, _D), lambda i: (0, 0)),
            pl.BlockSpec((8, _D), lambda i: (0, 0)),
        ],
        out_specs=[
            pl.BlockSpec((_BR, _D), lambda i: (i, 0)),
            pl.BlockSpec((8, _D), lambda i: (0, 0)),
        ],
        out_shape=[
            jax.ShapeDtypeStruct((_N, _D), jnp.float32),
            jax.ShapeDtypeStruct((8, _D), jnp.float32),
        ],
        scratch_shapes=[
            pltpu.VMEM((1, _D), jnp.float32),
            pltpu.VMEM((1, _D), jnp.float32),
        ],
    )(x, p[:_N], p[_NP:_NP + _N], W, b.reshape(1, _D),
      g.reshape(1, _D), be.reshape(1, _D), aff)


def _tc_apply(y, aff):
    def body(y_ref, aff_ref, o_ref):
        o_ref[...] = y_ref[...] * aff_ref[0] + aff_ref[1]

    return pl.pallas_call(
        body,
        grid=(_NB,),
        in_specs=[
            pl.BlockSpec((_BR, _D), lambda i: (i, 0)),
            pl.BlockSpec((8, _D), lambda i: (0, 0)),
        ],
        out_specs=pl.BlockSpec((_BR, _D), lambda i: (i, 0)),
        out_shape=jax.ShapeDtypeStruct((_N, _D), jnp.float32),
    )(y, aff)


def kernel(edge_index, node_attr, edge_attr,
           W1, b1, W2, b2, W3, b3, g1, be1, g2, be2, g3, be3):
    src = edge_index[0]
    dst = edge_index[1]
    x = node_attr
    aff = jnp.concatenate(
        [jnp.ones((1, _D), jnp.float32),
         jnp.zeros((7, _D), jnp.float32)], axis=0)
    for (W, b, g, be) in ((W1, b1, g1, be1),
                          (W2, b2, g2, be2),
                          (W3, b3, g3, be3)):
        p = _sc_message(src, dst, x, edge_attr, aff[:2].reshape(2 * _D))
        x, aff = _tc_linear(x, p, W, b, g, be, aff)
    return _tc_apply(x, aff)


# speedup vs baseline: 1.1340x; 1.1340x over previous
import functools

import jax
import jax.numpy as jnp
from jax import lax
from jax.experimental import pallas as pl
from jax.experimental.pallas import tpu as pltpu
from jax.experimental.pallas import tpu_sc as plsc

_N = 10000
_E = 320000
_D = 128
_NC = 2
_NS = 16
_NW = _NC * _NS
_EPW = _E // _NW
_CHUNK = 40
_NCHUNK = _EPW // _CHUNK
_NP = 10240
_RPT = _NP // _NS
_LANES = 16


def _sc_message(src, dst, x, ea, aff):
    mesh = plsc.VectorSubcoreMesh(core_axis_name="c", subcore_axis_name="s")

    @functools.partial(
        pl.kernel,
        mesh=mesh,
        out_type=jax.ShapeDtypeStruct((_NC * _NP, _D), jnp.float32),
        scratch_types=(
            [pltpu.VMEM_SHARED((_NP, _D), jnp.float32)]
            + [pltpu.VMEM((2 * _D,), jnp.float32)]
            + [pltpu.VMEM((_CHUNK,), jnp.int32)] * 4
            + [pltpu.VMEM((_CHUNK,), jnp.int32)] * 4
            + [pltpu.VMEM((_CHUNK, _D), jnp.float32)] * 8
            + [pltpu.SemaphoreType.DMA] * 21
        ),
    )
    def body(src_hbm, dst_hbm, x_hbm, ea_hbm, aff_hbm, out_hbm,
             acc, affb, sx0, sx1, sx2, sx3, dx0, dx1, dx2, dx3,
             xv0, xv1, xv2, xv3, ev0, ev1, ev2, ev3,
             p0, p1, p2, p3, q0, q1, q2, q3, g0, g1, g2, g3,
             f0, f1, f2, f3, s0, s1, s2, s3, zsem):
        sxs = (sx0, sx1, sx2, sx3)
        dxs = (dx0, dx1, dx2, dx3)
        xvs = (xv0, xv1, xv2, xv3)
        evs = (ev0, ev1, ev2, ev3)
        psems = (p0, p1, p2, p3)
        qsems = (q0, q1, q2, q3)
        gsems = (g0, g1, g2, g3)
        fsems = (f0, f1, f2, f3)
        ssems = (s0, s1, s2, s3)
        cid = lax.axis_index("c")
        sid = lax.axis_index("s")
        wid = sid * _NC + cid
        ebase = wid * _EPW

        zv = jnp.zeros((_LANES,), jnp.float32)

        @plsc.parallel_loop(0, _CHUNK, unroll=4)
        def _(r):
            for c in range(_D // _LANES):
                xv0[r, pl.ds(c * _LANES, _LANES)] = zv

        row0 = sid * _RPT
        for k in range(_RPT // _CHUNK):
            pltpu.async_copy(
                xv0, acc.at[pl.ds(row0 + k * _CHUNK, _CHUNK)], zsem)
        pltpu.sync_copy(aff_hbm, affb)
        for k in range(_RPT // _CHUNK):
            pltpu.make_async_copy(
                xv0, acc.at[pl.ds(0, _CHUNK)], zsem).wait()
        plsc.subcore_barrier()

        scv = [affb[pl.ds(c * _LANES, _LANES)] for c in range(_D // _LANES)]
        shv = [affb[pl.ds(_D + c * _LANES, _LANES)]
               for c in range(_D // _LANES)]

        def sfetch(j, b):
            pltpu.async_copy(
                src_hbm.at[pl.ds(ebase + j * _CHUNK, _CHUNK)], sxs[b], psems[b])

        def wait_sidx(b):
            pltpu.make_async_copy(
                src_hbm.at[pl.ds(0, _CHUNK)], sxs[b], psems[b]).wait()

        def dfetch(j, b):
            pltpu.async_copy(
                dst_hbm.at[pl.ds(ebase + j * _CHUNK, _CHUNK)], dxs[b], qsems[b])

        def wait_didx(b):
            pltpu.make_async_copy(
                dst_hbm.at[pl.ds(0, _CHUNK)], dxs[b], qsems[b]).wait()

        def issue(j, b):
            pltpu.async_copy(x_hbm.at[sxs[b]], xvs[b], gsems[b])
            pltpu.async_copy(
                ea_hbm.at[pl.ds(ebase + j * _CHUNK, _CHUNK)], evs[b], fsems[b])

        def wait_in(b):
            pltpu.make_async_copy(
                x_hbm.at[pl.ds(0, _CHUNK)], xvs[b], gsems[b]).wait()
            pltpu.make_async_copy(
                ea_hbm.at[pl.ds(0, _CHUNK)], evs[b], fsems[b]).wait()

        def wait_sc(b):
            pltpu.make_async_copy(
                evs[b], acc.at[pl.ds(0, _CHUNK)], ssems[b]).wait()

        def compute(b):
            xs, ms = xvs[b], evs[b]

            def rowf(r, carry):
                for c in range(_D // _LANES):
                    sl = pl.ds(c * _LANES, _LANES)
                    ms[r, sl] = jnp.maximum(
                        xs[r, sl] * scv[c] + (ms[r, sl] + shv[c]), 0.0)
                return carry

            lax.fori_loop(0, _CHUNK, rowf, 0)

        def scatter(j, b):
            pltpu.async_copy(evs[b], acc.at[dxs[b]], ssems[b], add=True)

        for b in range(4):
            sfetch(b, b)
        for b in range(2):
            dfetch(b, b)
            wait_sidx(b)
            issue(b, b)

        def quad(q, carry):
            for b in range(4):
                j = 4 * q + b
                nb = (b + 2) % 4
                wait_in(b)

                @pl.when(j + 4 < _NCHUNK)
                def _():
                    sfetch(j + 4, b)

                @pl.when(j >= 2)
                def _():
                    wait_sc(nb)

                @pl.when(j + 2 < _NCHUNK)
                def _():
                    wait_sidx(nb)
                    issue(j + 2, nb)
                    dfetch(j + 2, nb)

                compute(b)
                wait_didx(b)
                scatter(j, b)
            return carry

        lax.fori_loop(0, _NCHUNK // 4, quad, 0)

        for j in range(4 * (_NCHUNK // 4), _NCHUNK):
            b = j % 4
            wait_in(b)
            compute(b)
            wait_didx(b)
            scatter(j, b)
        for b in range(4):
            wait_sc(b)

        plsc.subcore_barrier()
        pltpu.sync_copy(acc.at[pl.ds(row0, _RPT)],
                        out_hbm.at[pl.ds(cid * _NP + row0, _RPT)])

    return body(src, dst, x, ea, aff)


_BR = 1000
_NB = _N // _BR


def _tc_linear(x, p, W, b, g, be, aff):
    def body(x_ref, p0_ref, p1_ref, w_ref, b_ref, g_ref, be_ref, aff_ref,
             y_ref, affo_ref, s1, s2):
        i = pl.program_id(0)
        h = (x_ref[...] * aff_ref[0] + aff_ref[1]
             + p0_ref[...] + p1_ref[...])
        y = jnp.dot(h, w_ref[...], preferred_element_type=jnp.float32)
        y = jnp.maximum(y + b_ref[...], 0.0)
        y_ref[...] = y
        c1 = jnp.sum(y, axis=0).reshape(1, _D)
        c2 = jnp.sum(y * y, axis=0).reshape(1, _D)

        @pl.when(i == 0)
        def _():
            s1[...] = c1
            s2[...] = c2
            affo_ref[...] = jnp.zeros((8, _D), jnp.float32)

        @pl.when(i > 0)
        def _():
            s1[...] = s1[...] + c1
            s2[...] = s2[...] + c2

        @pl.when(i == _NB - 1)
        def _():
            mean = s1[0] / _N
            var = s2[0] / _N - mean * mean
            scale = g_ref[0] * lax.rsqrt(var + 1e-5)
            shift = be_ref[0] - mean * scale
            affo_ref[0, :] = scale
            affo_ref[1, :] = shift

    return pl.pallas_call(
        body,
        grid=(_NB,),
        in_specs=[
            pl.BlockSpec((_BR, _D), lambda i: (i, 0)),
            pl.BlockSpec((_BR, _D), lambda i: (i, 0)),
            pl.BlockSpec((_BR, _D), lambda i: (i, 0)),
            pl.BlockSpec((_D, _D), lambda i: (0, 0)),
            pl.BlockSpec((1, _D), lambda i: (0, 0)),
            pl.BlockSpec((1, _D), lambda i: (0, 0)),
            pl.BlockSpec((1, _D), lambda i: (0, 0)),
            pl.BlockSpec((8, _D), lambda i: (0, 0)),
        ],
        out_specs=[
            pl.BlockSpec((_BR, _D), lambda i: (i, 0)),
            pl.BlockSpec((8, _D), lambda i: (0, 0)),
        ],
        out_shape=[
            jax.ShapeDtypeStruct((_N, _D), jnp.float32),
            jax.ShapeDtypeStruct((8, _D), jnp.float32),
        ],
        scratch_shapes=[
            pltpu.VMEM((1, _D), jnp.float32),
            pltpu.VMEM((1, _D), jnp.float32),
        ],
    )(x, p[:_N], p[_NP:_NP + _N], W, b.reshape(1, _D),
      g.reshape(1, _D), be.reshape(1, _D), aff)


def _tc_apply(y, aff):
    def body(y_ref, aff_ref, o_ref):
        o_ref[...] = y_ref[...] * aff_ref[0] + aff_ref[1]

    return pl.pallas_call(
        body,
        grid=(_NB,),
        in_specs=[
            pl.BlockSpec((_BR, _D), lambda i: (i, 0)),
            pl.BlockSpec((8, _D), lambda i: (0, 0)),
        ],
        out_specs=pl.BlockSpec((_BR, _D), lambda i: (i, 0)),
        out_shape=jax.ShapeDtypeStruct((_N, _D), jnp.float32),
    )(y, aff)


def kernel(edge_index, node_attr, edge_attr,
           W1, b1, W2, b2, W3, b3, g1, be1, g2, be2, g3, be3):
    src = edge_index[0]
    dst = edge_index[1]
    x = node_attr
    aff = jnp.concatenate(
        [jnp.ones((1, _D), jnp.float32),
         jnp.zeros((7, _D), jnp.float32)], axis=0)
    for (W, b, g, be) in ((W1, b1, g1, be1),
                          (W2, b2, g2, be2),
                          (W3, b3, g3, be3)):
        p = _sc_message(src, dst, x, edge_attr, aff[:2].reshape(2 * _D))
        x, aff = _tc_linear(x, p, W, b, g, be, aff)
    return _tc_apply(x, aff)
